# baseline (speedup 1.0000x reference)
import jax
import jax.numpy as jnp
from jax.experimental import pallas as pl
from jax.experimental.pallas import tpu as pltpu

B = 16
S = 336
N = 64
D = 64
DF = 128
E = 4
K = 2
L = 3
P = 96
SN = S * N
TT = 3584
NT = SN // TT
TTS = TT // N
NEG_INF = float("-inf")


def _mega_body(ti_ref, tg_ref, xn_ref, wg_ref, w1_ref, b1_ref, w2_ref, b2_ref,
               sw_ref, sb_ref, xout_ref, gates_ref, xcur_ref, gsum_ref,
               ei_ref, gf_ref):
    bi = pl.program_id(0)
    li = pl.program_id(1)
    ti = pl.program_id(2)
    iota_e = jax.lax.broadcasted_iota(jnp.int32, (1, E), 1)

    gate_in = gsum_ref[...] * jnp.float32(1.0 / SN)
    logits = jnp.dot(gate_in, wg_ref[li],
                     preferred_element_type=jnp.float32)
    m1 = jnp.max(logits)
    c_e0 = jnp.min(jnp.where(logits == m1, iota_e, E)).astype(jnp.int32)
    l2 = jnp.where(iota_e == c_e0, NEG_INF, logits)
    m2 = jnp.max(l2)
    c_e1 = jnp.min(jnp.where(l2 == m2, iota_e, E)).astype(jnp.int32)
    ex = jnp.exp(m2 - m1)
    c_g0 = 1.0 / (1.0 + ex)
    c_g1 = ex / (1.0 + ex)

    is_gate_step = jnp.logical_and(li > 0, ti == 0)

    @pl.when(is_gate_step)
    def _():
        ei_ref[0] = c_e0
        ei_ref[1] = c_e1
        gf_ref[0] = c_g0
        gf_ref[1] = c_g1
        idx = (li - 1) * 2
        gf_ref[2 + idx] = c_g0
        gf_ref[3 + idx] = c_g1
        ei_ref[2 + idx] = c_e0
        ei_ref[3 + idx] = c_e1

    use_cand = jnp.logical_and(is_gate_step, True)
    e0 = jnp.where(li == 0, ti_ref[bi, 0],
                   jnp.where(use_cand, c_e0, ei_ref[0]))
    e1 = jnp.where(li == 0, ti_ref[bi, 1],
                   jnp.where(use_cand, c_e1, ei_ref[1]))
    g0 = jnp.where(li == 0, tg_ref[bi, 0],
                   jnp.where(use_cand, c_g0, gf_ref[0]))
    g1 = jnp.where(li == 0, tg_ref[bi, 1],
                   jnp.where(use_cand, c_g1, gf_ref[1]))

    w1a = w1_ref[0, e0]
    w1b = w1_ref[0, e1]
    w2a = w2_ref[0, e0]
    w2b = w2_ref[0, e1]
    b1a = b1_ref[0, e0]
    b1b = b1_ref[0, e1]
    b2c = g0 * b2_ref[0, e0] + g1 * b2_ref[0, e1]

    def ffn(xt):
        h0 = jnp.maximum(
            jnp.dot(xt, w1a, preferred_element_type=jnp.float32) + b1a, 0.0)
        h1 = jnp.maximum(
            jnp.dot(xt, w1b, preferred_element_type=jnp.float32) + b1b, 0.0)
        y = (jnp.dot(h0, w2a, preferred_element_type=jnp.float32) * g0
             + jnp.dot(h1, w2b, preferred_element_type=jnp.float32) * g1)
        return xt + y + b2c

    sl = pl.ds(ti * TT, TT)

    @pl.when(li == 0)
    def _():
        xo = ffn(xn_ref[0] * sw_ref[...] + sb_ref[...])
        xcur_ref[sl, :] = xo
        colsum = jnp.sum(xo, axis=0, keepdims=True)
        gsum_ref[...] = jnp.where(ti == 0, colsum, gsum_ref[...] + colsum)

    @pl.when(jnp.logical_and(li > 0, li < L - 1))
    def _():
        xo = ffn(xcur_ref[sl, :])
        xcur_ref[sl, :] = xo
        colsum = jnp.sum(xo, axis=0, keepdims=True)
        gsum_ref[...] = jnp.where(ti == 0, colsum, gsum_ref[...] + colsum)

    @pl.when(li == L - 1)
    def _():
        xo = ffn(xcur_ref[sl, :])
        xout_ref[0] = jnp.transpose(xo.reshape(TTS, N, D), (1, 0, 2))

    @pl.when(jnp.logical_and(li == L - 1, ti == NT - 1))
    def _():
        rows = []
        for l in range(1, L):
            idx = (l - 1) * 2
            rows.append(jnp.where(iota_e == ei_ref[2 + idx], gf_ref[2 + idx], 0.0)
                        + jnp.where(iota_e == ei_ref[3 + idx], gf_ref[3 + idx], 0.0))
        gates_ref[0] = jnp.concatenate(rows, axis=1)


def _moe_stack(xn_col, w_gate, W1, b1, W2, b2, start_w, start_b, ti, tg):
    grid_spec = pltpu.PrefetchScalarGridSpec(
        num_scalar_prefetch=2,
        grid=(B, L, NT),
        in_specs=[
            pl.BlockSpec((1, TT, 1), lambda b, l, t, ii, gg: (b, t, 0)),
            pl.BlockSpec((L, D, E), lambda b, l, t, ii, gg: (0, 0, 0)),
            pl.BlockSpec((1, E, D, DF), lambda b, l, t, ii, gg: (l, 0, 0, 0)),
            pl.BlockSpec((1, E, 1, DF), lambda b, l, t, ii, gg: (l, 0, 0, 0)),
            pl.BlockSpec((1, E, DF, D), lambda b, l, t, ii, gg: (l, 0, 0, 0)),
            pl.BlockSpec((1, E, 1, D), lambda b, l, t, ii, gg: (l, 0, 0, 0)),
            pl.BlockSpec((1, D), lambda b, l, t, ii, gg: (0, 0)),
            pl.BlockSpec((1, D), lambda b, l, t, ii, gg: (0, 0)),
        ],
        out_specs=[
            pl.BlockSpec((1, N, TTS, D),
                         lambda b, l, t, ii, gg:
                         (b, 0, jnp.where(l == L - 1, t, 0), 0)),
            pl.BlockSpec((1, 1, (L - 1) * E), lambda b, l, t, ii, gg: (b, 0, 0)),
        ],
        scratch_shapes=[
            pltpu.VMEM((SN, D), jnp.float32),
            pltpu.VMEM((1, D), jnp.float32),
            pltpu.SMEM((8,), jnp.int32),
            pltpu.SMEM((8,), jnp.float32),
        ],
    )
    return pl.pallas_call(
        _mega_body,
        grid_spec=grid_spec,
        out_shape=[jax.ShapeDtypeStruct((B, N, S, D), jnp.float32),
                   jax.ShapeDtypeStruct((B, 1, (L - 1) * E), jnp.float32)],
        compiler_params=pltpu.CompilerParams(
            dimension_semantics=("arbitrary", "arbitrary", "arbitrary")),
    )(ti, tg, xn_col, w_gate, W1, b1.reshape(L, E, 1, DF), W2,
      b2.reshape(L, E, 1, D), start_w.reshape(1, D), start_b.reshape(1, D))


PKT = 3584
PNT = (S * D) // PKT


def _proj_body(x_ref, pw_ref, pb_ref, o_ref, acc_ref):
    k = pl.program_id(1)
    part = jnp.dot(x_ref[0], pw_ref[...], preferred_element_type=jnp.float32)

    @pl.when(k == 0)
    def _():
        acc_ref[...] = part

    @pl.when(k > 0)
    def _():
        acc_ref[...] += part

    @pl.when(k == PNT - 1)
    def _():
        o_ref[0] = jnp.transpose(acc_ref[...] + pb_ref[...], (1, 0))


def _projection(xt, proj_w, proj_b):
    return pl.pallas_call(
        _proj_body,
        grid=(B, PNT),
        in_specs=[
            pl.BlockSpec((1, N, PKT), lambda b, k: (b, 0, k)),
            pl.BlockSpec((PKT, P), lambda b, k: (k, 0)),
            pl.BlockSpec((1, P), lambda b, k: (0, 0)),
        ],
        out_specs=pl.BlockSpec((1, P, N), lambda b, k: (b, 0, 0)),
        out_shape=jax.ShapeDtypeStruct((B, P, N), jnp.float32),
        scratch_shapes=[pltpu.VMEM((N, P), jnp.float32)],
        compiler_params=pltpu.CompilerParams(
            dimension_semantics=("arbitrary", "arbitrary")),
    )(xt, proj_w, proj_b.reshape(1, P))


def _final_body(x_ref, w_ref, b_ref, o_ref):
    o_ref[...] = (jnp.dot(x_ref[...], w_ref[...], preferred_element_type=jnp.float32)
                  + b_ref[...])


def _final_head(x2, final_w, final_b):
    return pl.pallas_call(
        _final_body,
        out_shape=jax.ShapeDtypeStruct((B, P), jnp.float32),
    )(x2, final_w, final_b.reshape(1, P))


def _cv2(v):
    eps = 1e-10
    return jnp.var(v) / (jnp.mean(v) ** 2 + eps)


def kernel(x, start_w, start_b, w_gate, W1, b1, W2, b2, proj_w, proj_b,
           final_w, final_b):
    b = x.shape[0]
    mean = jnp.mean(x, axis=1, keepdims=True)
    std = jnp.sqrt(jnp.var(x, axis=1, keepdims=True) + 1e-5)
    xn = (x - mean) / std
    out = xn[..., None] * start_w + start_b
    gate_in0 = jnp.mean(out, axis=(1, 2))
    hsh = jax.nn.relu(jnp.einsum('bsnd,df->bsnf', out, W1[0, 0]) + b1[0, 0])
    keep = jnp.sum(hsh)

    logits0 = gate_in0 @ w_gate[0]
    top_logits0, top_idx0 = jax.lax.top_k(logits0, K)
    top_gates0 = jax.nn.softmax(top_logits0, axis=1)
    gates0 = jnp.zeros((b, E), dtype=jnp.float32).at[
        jnp.arange(b)[:, None], top_idx0].set(top_gates0)

    out_t, gates12 = _moe_stack(xn.reshape(b, SN, 1), w_gate, W1, b1, W2, b2,
                                start_w, start_b, top_idx0, top_gates0)

    balance_loss = jnp.asarray(0.0, dtype=jnp.float32)
    for l in range(L):
        g = gates0 if l == 0 else gates12[:, 0, (l - 1) * E:l * E]
        importance = jnp.sum(g, axis=0)
        load = jnp.sum((g > 0).astype(jnp.float32), axis=0)
        balance_loss = balance_loss + _cv2(importance) + _cv2(load)

    out2t = _projection(out_t.reshape(b, N, S * D), proj_w, proj_b)
    output = _final_head(out2t.reshape(b, P * N), final_w, final_b)
    balance_loss = balance_loss + keep * jnp.float32(1e-45)
    return output, balance_loss


# speedup vs baseline: 1.2473x; 1.2473x over previous
import functools

import jax
import jax.numpy as jnp
from jax.experimental import pallas as pl
from jax.experimental.pallas import tpu as pltpu

B = 16
S = 336
N = 64
D = 64
DF = 128
E = 4
K = 2
L = 3
P = 96
SN = S * N
TT = 7168
NT = SN // TT
TTS = TT // N


def _moe_body(idx_ref, gate_ref, xin_ref, w1a_ref, w1b_ref, b1a_ref, b1b_ref,
              w2a_ref, w2b_ref, b2a_ref, b2b_ref, xout_ref, *, transposed_out):
    bi = pl.program_id(0)
    g0 = gate_ref[bi, 0]
    g1 = gate_ref[bi, 1]
    x = xin_ref[0]
    h0 = jnp.maximum(
        jnp.dot(x, w1a_ref[0], preferred_element_type=jnp.float32) + b1a_ref[0], 0.0)
    h1 = jnp.maximum(
        jnp.dot(x, w1b_ref[0], preferred_element_type=jnp.float32) + b1b_ref[0], 0.0)
    y = (jnp.dot(h0, w2a_ref[0], preferred_element_type=jnp.float32) * g0
         + jnp.dot(h1, w2b_ref[0], preferred_element_type=jnp.float32) * g1)
    xo = x + y + (g0 * b2a_ref[0] + g1 * b2b_ref[0])
    if transposed_out:
        xout_ref[0] = jnp.transpose(xo.reshape(TTS, N, D), (1, 0, 2))
    else:
        xout_ref[0] = xo


def _moe_layer(out, w1l, b1l, w2l, b2l, top_idx, top_gates, *, last):
    body = functools.partial(_moe_body, transposed_out=last)
    if last:
        out_shape = jax.ShapeDtypeStruct((B, N, S, D), jnp.float32)
        out_spec = pl.BlockSpec((1, N, TTS, D), lambda b, t, ii, gg: (b, 0, t, 0))
    else:
        out_shape = jax.ShapeDtypeStruct((B, SN, D), jnp.float32)
        out_spec = pl.BlockSpec((1, TT, D), lambda b, t, ii, gg: (b, t, 0))
    grid_spec = pltpu.PrefetchScalarGridSpec(
        num_scalar_prefetch=2,
        grid=(B, NT),
        in_specs=[
            pl.BlockSpec((1, TT, D), lambda b, t, ii, gg: (b, t, 0)),
            pl.BlockSpec((1, D, DF), lambda b, t, ii, gg: (ii[b, 0], 0, 0)),
            pl.BlockSpec((1, D, DF), lambda b, t, ii, gg: (ii[b, 1], 0, 0)),
            pl.BlockSpec((1, 1, DF), lambda b, t, ii, gg: (ii[b, 0], 0, 0)),
            pl.BlockSpec((1, 1, DF), lambda b, t, ii, gg: (ii[b, 1], 0, 0)),
            pl.BlockSpec((1, DF, D), lambda b, t, ii, gg: (ii[b, 0], 0, 0)),
            pl.BlockSpec((1, DF, D), lambda b, t, ii, gg: (ii[b, 1], 0, 0)),
            pl.BlockSpec((1, 1, D), lambda b, t, ii, gg: (ii[b, 0], 0, 0)),
            pl.BlockSpec((1, 1, D), lambda b, t, ii, gg: (ii[b, 1], 0, 0)),
        ],
        out_specs=out_spec,
    )
    return pl.pallas_call(
        body,
        grid_spec=grid_spec,
        out_shape=out_shape,
        compiler_params=pltpu.CompilerParams(
            dimension_semantics=("arbitrary", "arbitrary")),
    )(top_idx, top_gates, out, w1l, w1l, b1l.reshape(E, 1, DF),
      b1l.reshape(E, 1, DF), w2l, w2l, b2l.reshape(E, 1, D), b2l.reshape(E, 1, D))


PKT = 7168
PNT = (S * D) // PKT


def _proj_body(x_ref, pw_ref, pb_ref, o_ref, acc_ref):
    k = pl.program_id(1)
    part = jnp.dot(x_ref[0], pw_ref[...], preferred_element_type=jnp.float32)

    @pl.when(k == 0)
    def _():
        acc_ref[...] = part

    @pl.when(k > 0)
    def _():
        acc_ref[...] += part

    @pl.when(k == PNT - 1)
    def _():
        o_ref[0] = jnp.transpose(acc_ref[...] + pb_ref[...], (1, 0))


def _projection(xt, proj_w, proj_b):
    return pl.pallas_call(
        _proj_body,
        grid=(B, PNT),
        in_specs=[
            pl.BlockSpec((1, N, PKT), lambda b, k: (b, 0, k)),
            pl.BlockSpec((PKT, P), lambda b, k: (k, 0)),
            pl.BlockSpec((1, P), lambda b, k: (0, 0)),
        ],
        out_specs=pl.BlockSpec((1, P, N), lambda b, k: (b, 0, 0)),
        out_shape=jax.ShapeDtypeStruct((B, P, N), jnp.float32),
        scratch_shapes=[pltpu.VMEM((N, P), jnp.float32)],
        compiler_params=pltpu.CompilerParams(
            dimension_semantics=("arbitrary", "arbitrary")),
    )(xt, proj_w, proj_b.reshape(1, P))


def _final_body(x_ref, w_ref, b_ref, o_ref):
    o_ref[...] = (jnp.dot(x_ref[...], w_ref[...], preferred_element_type=jnp.float32)
                  + b_ref[...])


def _final_head(x2, final_w, final_b):
    return pl.pallas_call(
        _final_body,
        out_shape=jax.ShapeDtypeStruct((B, P), jnp.float32),
    )(x2, final_w, final_b.reshape(1, P))


def kernel(x, start_w, start_b, w_gate, W1, b1, W2, b2, proj_w, proj_b,
           final_w, final_b):
    b = x.shape[0]
    mean = jnp.mean(x, axis=1, keepdims=True)
    std = jnp.sqrt(jnp.var(x, axis=1, keepdims=True) + 1e-5)
    xn = (x - mean) / std
    out = xn[..., None] * start_w + start_b
    balance_loss = jnp.asarray(0.0, dtype=jnp.float32)
    eps = 1e-10
    out_flat = out.reshape(b, SN, D)
    out4 = out
    keep = jnp.float32(0.0)
    for l in range(L):
        gate_in = jnp.mean(out4, axis=(1, 2))
        if l == 0:
            hsh = jax.nn.relu(jnp.einsum('bsnd,df->bsnf', out4, W1[0, 0]) + b1[0, 0])
            keep = jnp.sum(hsh)
        logits = gate_in @ w_gate[l]
        top_logits, top_idx = jax.lax.top_k(logits, K)
        top_gates = jax.nn.softmax(top_logits, axis=1)
        gates = jnp.zeros((b, E), dtype=jnp.float32).at[
            jnp.arange(b)[:, None], top_idx].set(top_gates)
        importance = jnp.sum(gates, axis=0)
        load = jnp.sum((gates > 0).astype(jnp.float32), axis=0)
        balance_loss = (balance_loss
                        + jnp.var(importance) / (jnp.mean(importance) ** 2 + eps)
                        + jnp.var(load) / (jnp.mean(load) ** 2 + eps))
        out_flat = _moe_layer(out_flat, W1[l], b1[l], W2[l], b2[l],
                              top_idx, top_gates, last=(l == L - 1))
        if l < L - 1:
            out4 = out_flat.reshape(b, S, N, D)
    out2t = _projection(out_flat.reshape(b, N, S * D), proj_w, proj_b)
    output = _final_head(out2t.reshape(b, P * N), final_w, final_b)
    balance_loss = balance_loss + keep * jnp.float32(1e-45)
    return output, balance_loss


# speedup vs baseline: 1.2856x; 1.0307x over previous
import functools

import jax
import jax.numpy as jnp
from jax.experimental import pallas as pl
from jax.experimental.pallas import tpu as pltpu

B = 16
S = 336
N = 64
D = 64
DF = 128
E = 4
K = 2
L = 3
P = 96
SN = S * N
TT = 10752
NT = SN // TT
TTS = TT // N


def _moe_body(idx_ref, gate_ref, xin_ref, w1a_ref, w1b_ref, b1a_ref, b1b_ref,
              w2a_ref, w2b_ref, b2a_ref, b2b_ref, xout_ref, *, transposed_out):
    bi = pl.program_id(0)
    g0 = gate_ref[bi, 0]
    g1 = gate_ref[bi, 1]
    x = xin_ref[0]
    h0 = jnp.maximum(
        jnp.dot(x, w1a_ref[0], preferred_element_type=jnp.float32) + b1a_ref[0], 0.0)
    h1 = jnp.maximum(
        jnp.dot(x, w1b_ref[0], preferred_element_type=jnp.float32) + b1b_ref[0], 0.0)
    y = (jnp.dot(h0, w2a_ref[0], preferred_element_type=jnp.float32) * g0
         + jnp.dot(h1, w2b_ref[0], preferred_element_type=jnp.float32) * g1)
    xo = x + y + (g0 * b2a_ref[0] + g1 * b2b_ref[0])
    if transposed_out:
        xout_ref[0] = jnp.transpose(xo.reshape(TTS, N, D), (1, 0, 2))
    else:
        xout_ref[0] = xo


def _moe_layer(out, w1l, b1l, w2l, b2l, top_idx, top_gates, *, last):
    body = functools.partial(_moe_body, transposed_out=last)
    if last:
        out_shape = jax.ShapeDtypeStruct((B, N, S, D), jnp.float32)
        out_spec = pl.BlockSpec((1, N, TTS, D), lambda b, t, ii, gg: (b, 0, t, 0))
    else:
        out_shape = jax.ShapeDtypeStruct((B, SN, D), jnp.float32)
        out_spec = pl.BlockSpec((1, TT, D), lambda b, t, ii, gg: (b, t, 0))
    grid_spec = pltpu.PrefetchScalarGridSpec(
        num_scalar_prefetch=2,
        grid=(B, NT),
        in_specs=[
            pl.BlockSpec((1, TT, D), lambda b, t, ii, gg: (b, t, 0)),
            pl.BlockSpec((1, D, DF), lambda b, t, ii, gg: (ii[b, 0], 0, 0)),
            pl.BlockSpec((1, D, DF), lambda b, t, ii, gg: (ii[b, 1], 0, 0)),
            pl.BlockSpec((1, 1, DF), lambda b, t, ii, gg: (ii[b, 0], 0, 0)),
            pl.BlockSpec((1, 1, DF), lambda b, t, ii, gg: (ii[b, 1], 0, 0)),
            pl.BlockSpec((1, DF, D), lambda b, t, ii, gg: (ii[b, 0], 0, 0)),
            pl.BlockSpec((1, DF, D), lambda b, t, ii, gg: (ii[b, 1], 0, 0)),
            pl.BlockSpec((1, 1, D), lambda b, t, ii, gg: (ii[b, 0], 0, 0)),
            pl.BlockSpec((1, 1, D), lambda b, t, ii, gg: (ii[b, 1], 0, 0)),
        ],
        out_specs=out_spec,
    )
    return pl.pallas_call(
        body,
        grid_spec=grid_spec,
        out_shape=out_shape,
        compiler_params=pltpu.CompilerParams(
            dimension_semantics=("arbitrary", "arbitrary")),
    )(top_idx, top_gates, out, w1l, w1l, b1l.reshape(E, 1, DF),
      b1l.reshape(E, 1, DF), w2l, w2l, b2l.reshape(E, 1, D), b2l.reshape(E, 1, D))


PKT = 10752
PNT = (S * D) // PKT


def _proj_body(x_ref, pw_ref, pb_ref, o_ref, acc_ref):
    k = pl.program_id(1)
    part = jnp.dot(x_ref[0], pw_ref[...], preferred_element_type=jnp.float32)

    @pl.when(k == 0)
    def _():
        acc_ref[...] = part

    @pl.when(k > 0)
    def _():
        acc_ref[...] += part

    @pl.when(k == PNT - 1)
    def _():
        o_ref[0] = jnp.transpose(acc_ref[...] + pb_ref[...], (1, 0))


def _projection(xt, proj_w, proj_b):
    return pl.pallas_call(
        _proj_body,
        grid=(B, PNT),
        in_specs=[
            pl.BlockSpec((1, N, PKT), lambda b, k: (b, 0, k)),
            pl.BlockSpec((PKT, P), lambda b, k: (k, 0)),
            pl.BlockSpec((1, P), lambda b, k: (0, 0)),
        ],
        out_specs=pl.BlockSpec((1, P, N), lambda b, k: (b, 0, 0)),
        out_shape=jax.ShapeDtypeStruct((B, P, N), jnp.float32),
        scratch_shapes=[pltpu.VMEM((N, P), jnp.float32)],
        compiler_params=pltpu.CompilerParams(
            dimension_semantics=("arbitrary", "arbitrary")),
    )(xt, proj_w, proj_b.reshape(1, P))


def _final_body(x_ref, w_ref, b_ref, o_ref):
    o_ref[...] = (jnp.dot(x_ref[...], w_ref[...], preferred_element_type=jnp.float32)
                  + b_ref[...])


def _final_head(x2, final_w, final_b):
    return pl.pallas_call(
        _final_body,
        out_shape=jax.ShapeDtypeStruct((B, P), jnp.float32),
    )(x2, final_w, final_b.reshape(1, P))


def kernel(x, start_w, start_b, w_gate, W1, b1, W2, b2, proj_w, proj_b,
           final_w, final_b):
    b = x.shape[0]
    mean = jnp.mean(x, axis=1, keepdims=True)
    std = jnp.sqrt(jnp.var(x, axis=1, keepdims=True) + 1e-5)
    xn = (x - mean) / std
    out = xn[..., None] * start_w + start_b
    balance_loss = jnp.asarray(0.0, dtype=jnp.float32)
    eps = 1e-10
    out_flat = out.reshape(b, SN, D)
    out4 = out
    keep = jnp.float32(0.0)
    for l in range(L):
        gate_in = jnp.mean(out4, axis=(1, 2))
        if l == 0:
            hsh = jax.nn.relu(jnp.einsum('bsnd,df->bsnf', out4, W1[0, 0]) + b1[0, 0])
            keep = jnp.sum(hsh)
        logits = gate_in @ w_gate[l]
        top_logits, top_idx = jax.lax.top_k(logits, K)
        top_gates = jax.nn.softmax(top_logits, axis=1)
        gates = jnp.zeros((b, E), dtype=jnp.float32).at[
            jnp.arange(b)[:, None], top_idx].set(top_gates)
        importance = jnp.sum(gates, axis=0)
        load = jnp.sum((gates > 0).astype(jnp.float32), axis=0)
        balance_loss = (balance_loss
                        + jnp.var(importance) / (jnp.mean(importance) ** 2 + eps)
                        + jnp.var(load) / (jnp.mean(load) ** 2 + eps))
        out_flat = _moe_layer(out_flat, W1[l], b1[l], W2[l], b2[l],
                              top_idx, top_gates, last=(l == L - 1))
        if l < L - 1:
            out4 = out_flat.reshape(b, S, N, D)
    out2t = _projection(out_flat.reshape(b, N, S * D), proj_w, proj_b)
    output = _final_head(out2t.reshape(b, P * N), final_w, final_b)
    balance_loss = balance_loss + keep * jnp.float32(1e-45)
    return output, balance_loss


# speedup vs baseline: 1.3575x; 1.0559x over previous
import jax
import jax.numpy as jnp
from jax.experimental import pallas as pl
from jax.experimental.pallas import tpu as pltpu

B = 16
S = 336
N = 64
D = 64
DF = 128
E = 4
K = 2
L = 3
P = 96
SN = S * N
TT = 10752
NT = SN // TT
TTS = TT // N
NEG_INF = float("-inf")


def _l0_body(idx_ref, gate_ref, xin_ref, w1a_ref, w1b_ref, b1a_ref, b1b_ref,
             w2a_ref, w2b_ref, b2a_ref, b2b_ref, xout_ref, gsum_ref):
    bi = pl.program_id(0)
    t = pl.program_id(1)
    g0 = gate_ref[bi, 0]
    g1 = gate_ref[bi, 1]
    x = xin_ref[0]
    h0 = jnp.maximum(
        jnp.dot(x, w1a_ref[0], preferred_element_type=jnp.float32) + b1a_ref[0], 0.0)
    h1 = jnp.maximum(
        jnp.dot(x, w1b_ref[0], preferred_element_type=jnp.float32) + b1b_ref[0], 0.0)
    y = (jnp.dot(h0, w2a_ref[0], preferred_element_type=jnp.float32) * g0
         + jnp.dot(h1, w2b_ref[0], preferred_element_type=jnp.float32) * g1)
    xo = x + y + (g0 * b2a_ref[0] + g1 * b2b_ref[0])
    xout_ref[0] = xo
    colsum = jnp.sum(xo, axis=0, keepdims=True)
    gsum_ref[0] = jnp.where(t == 0, colsum, gsum_ref[0] + colsum)


def _moe_layer0(out, w1l, b1l, w2l, b2l, top_idx, top_gates):
    grid_spec = pltpu.PrefetchScalarGridSpec(
        num_scalar_prefetch=2,
        grid=(B, NT),
        in_specs=[
            pl.BlockSpec((1, TT, D), lambda b, t, ii, gg: (b, t, 0)),
            pl.BlockSpec((1, D, DF), lambda b, t, ii, gg: (ii[b, 0], 0, 0)),
            pl.BlockSpec((1, D, DF), lambda b, t, ii, gg: (ii[b, 1], 0, 0)),
            pl.BlockSpec((1, 1, DF), lambda b, t, ii, gg: (ii[b, 0], 0, 0)),
            pl.BlockSpec((1, 1, DF), lambda b, t, ii, gg: (ii[b, 1], 0, 0)),
            pl.BlockSpec((1, DF, D), lambda b, t, ii, gg: (ii[b, 0], 0, 0)),
            pl.BlockSpec((1, DF, D), lambda b, t, ii, gg: (ii[b, 1], 0, 0)),
            pl.BlockSpec((1, 1, D), lambda b, t, ii, gg: (ii[b, 0], 0, 0)),
            pl.BlockSpec((1, 1, D), lambda b, t, ii, gg: (ii[b, 1], 0, 0)),
        ],
        out_specs=[
            pl.BlockSpec((1, TT, D), lambda b, t, ii, gg: (b, t, 0)),
            pl.BlockSpec((1, 1, D), lambda b, t, ii, gg: (b, 0, 0)),
        ],
    )
    return pl.pallas_call(
        _l0_body,
        grid_spec=grid_spec,
        out_shape=[jax.ShapeDtypeStruct((B, SN, D), jnp.float32),
                   jax.ShapeDtypeStruct((B, 1, D), jnp.float32)],
        compiler_params=pltpu.CompilerParams(
            dimension_semantics=("arbitrary", "arbitrary")),
    )(top_idx, top_gates, out, w1l, w1l, b1l.reshape(E, 1, DF),
      b1l.reshape(E, 1, DF), w2l, w2l, b2l.reshape(E, 1, D), b2l.reshape(E, 1, D))


def _sel4(e, full_ref, li):
    r = full_ref[li, 3]
    for idx in (2, 1, 0):
        r = jnp.where(e == idx, full_ref[li, idx], r)
    return r


def _l12_body(idx_ref, gate_ref, xin_ref, wg_ref, w1a_ref, w1b_ref, b1a_ref,
              b1b_ref, w2a_ref, w2b_ref, b2a_ref, b2b_ref, w1f_ref, b1f_ref,
              w2f_ref, b2f_ref, xout_ref, g2_ref, xcur_ref, gsum_ref,
              ei_ref, gf_ref):
    bi = pl.program_id(0)
    lp = pl.program_id(1)
    t = pl.program_id(2)
    iota_e = jax.lax.broadcasted_iota(jnp.int32, (1, E), 1)

    gate_in = gsum_ref[...] * jnp.float32(1.0 / SN)
    logits = jnp.dot(gate_in, wg_ref[0], preferred_element_type=jnp.float32)
    m1 = jnp.max(logits)
    c_e0 = jnp.min(jnp.where(logits == m1, iota_e, E)).astype(jnp.int32)
    lg2 = jnp.where(iota_e == c_e0, NEG_INF, logits)
    m2 = jnp.max(lg2)
    c_e1 = jnp.min(jnp.where(lg2 == m2, iota_e, E)).astype(jnp.int32)
    ex = jnp.exp(m2 - m1)
    c_g0 = 1.0 / (1.0 + ex)
    c_g1 = ex / (1.0 + ex)

    is_gate_step = jnp.logical_and(lp == 1, t == 0)

    @pl.when(is_gate_step)
    def _():
        ei_ref[0] = c_e0
        ei_ref[1] = c_e1
        gf_ref[0] = c_g0
        gf_ref[1] = c_g1

    e0 = jnp.where(t == 0, c_e0, ei_ref[0])
    e1 = jnp.where(t == 0, c_e1, ei_ref[1])
    gg0 = jnp.where(t == 0, c_g0, gf_ref[0])
    gg1 = jnp.where(t == 0, c_g1, gf_ref[1])

    g0 = jnp.where(lp == 0, gate_ref[bi, 0], gg0)
    g1 = jnp.where(lp == 0, gate_ref[bi, 1], gg1)
    w1a = jnp.where(lp == 0, w1a_ref[0], _sel4(e0, w1f_ref, 1))
    w1b = jnp.where(lp == 0, w1b_ref[0], _sel4(e1, w1f_ref, 1))
    w2a = jnp.where(lp == 0, w2a_ref[0], _sel4(e0, w2f_ref, 1))
    w2b = jnp.where(lp == 0, w2b_ref[0], _sel4(e1, w2f_ref, 1))
    b1a = jnp.where(lp == 0, b1a_ref[0], _sel4(e0, b1f_ref, 1))
    b1b = jnp.where(lp == 0, b1b_ref[0], _sel4(e1, b1f_ref, 1))
    b2a = jnp.where(lp == 0, b2a_ref[0], _sel4(e0, b2f_ref, 1))
    b2b = jnp.where(lp == 0, b2b_ref[0], _sel4(e1, b2f_ref, 1))

    sl = pl.ds(t * TT, TT)
    x = jnp.where(lp == 0, xin_ref[0], xcur_ref[sl, :])
    h0 = jnp.maximum(jnp.dot(x, w1a, preferred_element_type=jnp.float32) + b1a, 0.0)
    h1 = jnp.maximum(jnp.dot(x, w1b, preferred_element_type=jnp.float32) + b1b, 0.0)
    y = (jnp.dot(h0, w2a, preferred_element_type=jnp.float32) * g0
         + jnp.dot(h1, w2b, preferred_element_type=jnp.float32) * g1)
    xo = x + y + (g0 * b2a + g1 * b2b)
    xcur_ref[sl, :] = xo
    colsum = jnp.sum(xo, axis=0, keepdims=True)
    gsum_ref[...] = jnp.where(t == 0, colsum, gsum_ref[...] + colsum)

    @pl.when(lp == 1)
    def _():
        xout_ref[0] = jnp.transpose(xo.reshape(TTS, N, D), (1, 0, 2))

    @pl.when(jnp.logical_and(lp == 1, t == NT - 1))
    def _():
        g2_ref[0] = (jnp.where(iota_e == ei_ref[0], gf_ref[0], 0.0)
                     + jnp.where(iota_e == ei_ref[1], gf_ref[1], 0.0))


def _moe_layers12(act1, wg2, w1l1, b1l1, w2l1, b2l1, W1f, b1f, W2f, b2f,
                  top_idx, top_gates):
    grid_spec = pltpu.PrefetchScalarGridSpec(
        num_scalar_prefetch=2,
        grid=(B, 2, NT),
        in_specs=[
            pl.BlockSpec((1, TT, D),
                         lambda b, l, t, ii, gg:
                         (b, jnp.where(l == 0, t, NT - 1), 0)),
            pl.BlockSpec((1, D, E), lambda b, l, t, ii, gg: (0, 0, 0)),
            pl.BlockSpec((1, D, DF), lambda b, l, t, ii, gg: (ii[b, 0], 0, 0)),
            pl.BlockSpec((1, D, DF), lambda b, l, t, ii, gg: (ii[b, 1], 0, 0)),
            pl.BlockSpec((1, 1, DF), lambda b, l, t, ii, gg: (ii[b, 0], 0, 0)),
            pl.BlockSpec((1, 1, DF), lambda b, l, t, ii, gg: (ii[b, 1], 0, 0)),
            pl.BlockSpec((1, DF, D), lambda b, l, t, ii, gg: (ii[b, 0], 0, 0)),
            pl.BlockSpec((1, DF, D), lambda b, l, t, ii, gg: (ii[b, 1], 0, 0)),
            pl.BlockSpec((1, 1, D), lambda b, l, t, ii, gg: (ii[b, 0], 0, 0)),
            pl.BlockSpec((1, 1, D), lambda b, l, t, ii, gg: (ii[b, 1], 0, 0)),
            pl.BlockSpec((2, E, D, DF), lambda b, l, t, ii, gg: (0, 0, 0, 0)),
            pl.BlockSpec((2, E, 1, DF), lambda b, l, t, ii, gg: (0, 0, 0, 0)),
            pl.BlockSpec((2, E, DF, D), lambda b, l, t, ii, gg: (0, 0, 0, 0)),
            pl.BlockSpec((2, E, 1, D), lambda b, l, t, ii, gg: (0, 0, 0, 0)),
        ],
        out_specs=[
            pl.BlockSpec((1, N, TTS, D),
                         lambda b, l, t, ii, gg:
                         (b, 0, jnp.where(l == 1, t, 0), 0)),
            pl.BlockSpec((1, 1, E), lambda b, l, t, ii, gg: (b, 0, 0)),
        ],
        scratch_shapes=[
            pltpu.VMEM((SN, D), jnp.float32),
            pltpu.VMEM((1, D), jnp.float32),
            pltpu.SMEM((2,), jnp.int32),
            pltpu.SMEM((2,), jnp.float32),
        ],
    )
    return pl.pallas_call(
        _l12_body,
        grid_spec=grid_spec,
        out_shape=[jax.ShapeDtypeStruct((B, N, S, D), jnp.float32),
                   jax.ShapeDtypeStruct((B, 1, E), jnp.float32)],
        compiler_params=pltpu.CompilerParams(
            dimension_semantics=("arbitrary", "arbitrary", "arbitrary")),
    )(top_idx, top_gates, act1, wg2.reshape(1, D, E), w1l1, w1l1,
      b1l1.reshape(E, 1, DF), b1l1.reshape(E, 1, DF), w2l1, w2l1,
      b2l1.reshape(E, 1, D), b2l1.reshape(E, 1, D),
      W1f, b1f.reshape(2, E, 1, DF), W2f, b2f.reshape(2, E, 1, D))


PKT = 10752
PNT = (S * D) // PKT


def _proj_body(x_ref, pw_ref, pb_ref, o_ref, acc_ref):
    k = pl.program_id(1)
    part = jnp.dot(x_ref[0], pw_ref[...], preferred_element_type=jnp.float32)

    @pl.when(k == 0)
    def _():
        acc_ref[...] = part

    @pl.when(k > 0)
    def _():
        acc_ref[...] += part

    @pl.when(k == PNT - 1)
    def _():
        o_ref[0] = jnp.transpose(acc_ref[...] + pb_ref[...], (1, 0))


def _projection(xt, proj_w, proj_b):
    return pl.pallas_call(
        _proj_body,
        grid=(B, PNT),
        in_specs=[
            pl.BlockSpec((1, N, PKT), lambda b, k: (b, 0, k)),
            pl.BlockSpec((PKT, P), lambda b, k: (k, 0)),
            pl.BlockSpec((1, P), lambda b, k: (0, 0)),
        ],
        out_specs=pl.BlockSpec((1, P, N), lambda b, k: (b, 0, 0)),
        out_shape=jax.ShapeDtypeStruct((B, P, N), jnp.float32),
        scratch_shapes=[pltpu.VMEM((N, P), jnp.float32)],
        compiler_params=pltpu.CompilerParams(
            dimension_semantics=("arbitrary", "arbitrary")),
    )(xt, proj_w, proj_b.reshape(1, P))


def _final_body(x_ref, w_ref, b_ref, o_ref):
    o_ref[...] = (jnp.dot(x_ref[...], w_ref[...], preferred_element_type=jnp.float32)
                  + b_ref[...])


def _final_head(x2, final_w, final_b):
    return pl.pallas_call(
        _final_body,
        out_shape=jax.ShapeDtypeStruct((B, P), jnp.float32),
    )(x2, final_w, final_b.reshape(1, P))


def _cv2(v):
    eps = 1e-10
    return jnp.var(v) / (jnp.mean(v) ** 2 + eps)


def _gate_chain(gate_in, w_gate_l, b):
    logits = gate_in @ w_gate_l
    top_logits, top_idx = jax.lax.top_k(logits, K)
    top_gates = jax.nn.softmax(top_logits, axis=1)
    gates = jnp.zeros((b, E), dtype=jnp.float32).at[
        jnp.arange(b)[:, None], top_idx].set(top_gates)
    return top_idx, top_gates, gates


def kernel(x, start_w, start_b, w_gate, W1, b1, W2, b2, proj_w, proj_b,
           final_w, final_b):
    b = x.shape[0]
    mean = jnp.mean(x, axis=1, keepdims=True)
    std = jnp.sqrt(jnp.var(x, axis=1, keepdims=True) + 1e-5)
    xn = (x - mean) / std
    out = xn[..., None] * start_w + start_b
    gate_in0 = jnp.mean(out, axis=(1, 2))
    hsh = jax.nn.relu(jnp.einsum('bsnd,df->bsnf', out, W1[0, 0]) + b1[0, 0])
    keep = jnp.sum(hsh)

    ti0, tg0, gates0 = _gate_chain(gate_in0, w_gate[0], b)
    act1, gsum1 = _moe_layer0(out.reshape(b, SN, D), W1[0], b1[0], W2[0], b2[0],
                              ti0, tg0)
    gate_in1 = gsum1[:, 0, :] * jnp.float32(1.0 / SN)
    ti1, tg1, gates1 = _gate_chain(gate_in1, w_gate[1], b)
    out_t, gates2r = _moe_layers12(act1, w_gate[2], W1[1], b1[1], W2[1], b2[1],
                                   W1[1:], b1[1:], W2[1:], b2[1:], ti1, tg1)

    balance_loss = jnp.asarray(0.0, dtype=jnp.float32)
    for g in [gates0, gates1, gates2r[:, 0, :]]:
        importance = jnp.sum(g, axis=0)
        load = jnp.sum((g > 0).astype(jnp.float32), axis=0)
        balance_loss = balance_loss + _cv2(importance) + _cv2(load)

    out2t = _projection(out_t.reshape(b, N, S * D), proj_w, proj_b)
    output = _final_head(out2t.reshape(b, P * N), final_w, final_b)
    balance_loss = balance_loss + keep * jnp.float32(1e-45)
    return output, balance_loss
